# Initial kernel scaffold; baseline (speedup 1.0000x reference)
import jax
import jax.numpy as jnp
from jax.experimental import pallas as pl


def kernel(hidden_states, W, b):
    raise NotImplementedError("write your pallas kernel here")


# speedup vs baseline: 4.1581x; 4.1581x over previous
import jax
import jax.numpy as jnp
from jax.experimental import pallas as pl

_NUM_TOKENS = 32768
_HIDDEN = 768
_NUM_EXPERTS = 8
_BLOCK = 2048


def _router_block_kernel(h_ref, wt_ref, b_ref, sparse_ref, logits_ref):
    h = h_ref[...]
    wt = wt_ref[...]
    logits = jax.lax.dot_general(
        h, wt, (((1,), (0,)), ((), ())), preferred_element_type=jnp.float32
    ) + b_ref[...]
    logits_ref[...] = logits

    m = jnp.max(logits, axis=-1, keepdims=True)
    e = jnp.exp(logits - m)
    p = e / jnp.sum(e, axis=-1, keepdims=True)

    idx = jax.lax.broadcasted_iota(jnp.int32, p.shape, 1)
    m1 = jnp.max(p, axis=-1, keepdims=True)
    i1 = jnp.min(jnp.where(p == m1, idx, _NUM_EXPERTS), axis=-1, keepdims=True)
    p_rest = jnp.where(idx == i1, -jnp.inf, p)
    m2 = jnp.max(p_rest, axis=-1, keepdims=True)
    i2 = jnp.min(jnp.where(p_rest == m2, idx, _NUM_EXPERTS), axis=-1, keepdims=True)

    denom = m1 + m2
    w1 = m1 / denom
    w2 = m2 / denom
    sparse_ref[...] = jnp.where(idx == i1, w1, jnp.where(idx == i2, w2, 0.0))


def kernel(hidden_states, W, b):
    n_tokens = hidden_states.shape[0]
    wt = W.T
    b2 = b.reshape(1, _NUM_EXPERTS)
    grid = (n_tokens // _BLOCK,)
    sparse, logits = pl.pallas_call(
        _router_block_kernel,
        grid=grid,
        in_specs=[
            pl.BlockSpec((_BLOCK, _HIDDEN), lambda i: (i, 0)),
            pl.BlockSpec((_HIDDEN, _NUM_EXPERTS), lambda i: (0, 0)),
            pl.BlockSpec((1, _NUM_EXPERTS), lambda i: (0, 0)),
        ],
        out_specs=[
            pl.BlockSpec((_BLOCK, _NUM_EXPERTS), lambda i: (i, 0)),
            pl.BlockSpec((_BLOCK, _NUM_EXPERTS), lambda i: (i, 0)),
        ],
        out_shape=[
            jax.ShapeDtypeStruct((n_tokens, _NUM_EXPERTS), jnp.float32),
            jax.ShapeDtypeStruct((n_tokens, _NUM_EXPERTS), jnp.float32),
        ],
    )(hidden_states, wt, b2)
    return (sparse, logits)


# speedup vs baseline: 4.3792x; 1.0532x over previous
import jax
import jax.numpy as jnp
from jax.experimental import pallas as pl

_NUM_TOKENS = 32768
_HIDDEN = 768
_NUM_EXPERTS = 8
_BLOCK = 2048


def _router_block_kernel(h_ref, wt_ref, b_ref, tri_ref, sparse_ref, logits_ref):
    h = h_ref[...]
    wt = wt_ref[...]
    logits = jax.lax.dot_general(
        h, wt, (((1,), (0,)), ((), ())), preferred_element_type=jnp.float32
    ) + b_ref[...]
    logits_ref[...] = logits

    m = jnp.max(logits, axis=-1, keepdims=True)
    e = jnp.exp(logits - m)
    p = e / jnp.sum(e, axis=-1, keepdims=True)

    tri = tri_ref[...]
    m1 = jnp.max(p, axis=-1, keepdims=True)
    is1 = (p == m1).astype(jnp.float32)
    before1 = jax.lax.dot_general(
        is1, tri, (((1,), (0,)), ((), ())), preferred_element_type=jnp.float32
    )
    mask1 = (p == m1) & (before1 == 0.0)

    p_rest = jnp.where(mask1, -jnp.inf, p)
    m2 = jnp.max(p_rest, axis=-1, keepdims=True)
    is2 = (p_rest == m2).astype(jnp.float32)
    before2 = jax.lax.dot_general(
        is2, tri, (((1,), (0,)), ((), ())), preferred_element_type=jnp.float32
    )
    mask2 = (p_rest == m2) & (before2 == 0.0)

    denom = m1 + m2
    w1 = m1 / denom
    w2 = m2 / denom
    sparse_ref[...] = jnp.where(mask1, w1, jnp.where(mask2, w2, 0.0))


def kernel(hidden_states, W, b):
    n_tokens = hidden_states.shape[0]
    wt = W.T
    b2 = b.reshape(1, _NUM_EXPERTS)
    tri = jnp.triu(jnp.ones((_NUM_EXPERTS, _NUM_EXPERTS), jnp.float32), k=1)
    grid = (n_tokens // _BLOCK,)
    sparse, logits = pl.pallas_call(
        _router_block_kernel,
        grid=grid,
        in_specs=[
            pl.BlockSpec((_BLOCK, _HIDDEN), lambda i: (i, 0)),
            pl.BlockSpec((_HIDDEN, _NUM_EXPERTS), lambda i: (0, 0)),
            pl.BlockSpec((1, _NUM_EXPERTS), lambda i: (0, 0)),
            pl.BlockSpec((_NUM_EXPERTS, _NUM_EXPERTS), lambda i: (0, 0)),
        ],
        out_specs=[
            pl.BlockSpec((_BLOCK, _NUM_EXPERTS), lambda i: (i, 0)),
            pl.BlockSpec((_BLOCK, _NUM_EXPERTS), lambda i: (i, 0)),
        ],
        out_shape=[
            jax.ShapeDtypeStruct((n_tokens, _NUM_EXPERTS), jnp.float32),
            jax.ShapeDtypeStruct((n_tokens, _NUM_EXPERTS), jnp.float32),
        ],
    )(hidden_states, wt, b2, tri)
    return (sparse, logits)


# speedup vs baseline: 4.5722x; 1.0441x over previous
import jax
import jax.numpy as jnp
from jax.experimental import pallas as pl

_NUM_TOKENS = 32768
_HIDDEN = 768
_NUM_EXPERTS = 8
_BLOCK = 4096


def _router_block_kernel(h_ref, wt_ref, b_ref, tri_ref, sparse_ref, logits_ref):
    h = h_ref[...]
    wt = wt_ref[...]
    logits = jax.lax.dot_general(
        h, wt, (((1,), (0,)), ((), ())), preferred_element_type=jnp.float32
    ) + b_ref[...]
    logits_ref[...] = logits

    m = jnp.max(logits, axis=-1, keepdims=True)
    e = jnp.exp(logits - m)
    p = e / jnp.sum(e, axis=-1, keepdims=True)

    tri = tri_ref[...]
    m1 = jnp.max(p, axis=-1, keepdims=True)
    is1 = (p == m1).astype(jnp.float32)
    before1 = jax.lax.dot_general(
        is1, tri, (((1,), (0,)), ((), ())), preferred_element_type=jnp.float32
    )
    mask1 = (p == m1) & (before1 == 0.0)

    p_rest = jnp.where(mask1, -jnp.inf, p)
    m2 = jnp.max(p_rest, axis=-1, keepdims=True)
    is2 = (p_rest == m2).astype(jnp.float32)
    before2 = jax.lax.dot_general(
        is2, tri, (((1,), (0,)), ((), ())), preferred_element_type=jnp.float32
    )
    mask2 = (p_rest == m2) & (before2 == 0.0)

    denom = m1 + m2
    w1 = m1 / denom
    w2 = m2 / denom
    sparse_ref[...] = jnp.where(mask1, w1, jnp.where(mask2, w2, 0.0))


def kernel(hidden_states, W, b):
    n_tokens = hidden_states.shape[0]
    wt = W.T
    b2 = b.reshape(1, _NUM_EXPERTS)
    tri = jnp.triu(jnp.ones((_NUM_EXPERTS, _NUM_EXPERTS), jnp.float32), k=1)
    grid = (n_tokens // _BLOCK,)
    sparse, logits = pl.pallas_call(
        _router_block_kernel,
        grid=grid,
        in_specs=[
            pl.BlockSpec((_BLOCK, _HIDDEN), lambda i: (i, 0)),
            pl.BlockSpec((_HIDDEN, _NUM_EXPERTS), lambda i: (0, 0)),
            pl.BlockSpec((1, _NUM_EXPERTS), lambda i: (0, 0)),
            pl.BlockSpec((_NUM_EXPERTS, _NUM_EXPERTS), lambda i: (0, 0)),
        ],
        out_specs=[
            pl.BlockSpec((_BLOCK, _NUM_EXPERTS), lambda i: (i, 0)),
            pl.BlockSpec((_BLOCK, _NUM_EXPERTS), lambda i: (i, 0)),
        ],
        out_shape=[
            jax.ShapeDtypeStruct((n_tokens, _NUM_EXPERTS), jnp.float32),
            jax.ShapeDtypeStruct((n_tokens, _NUM_EXPERTS), jnp.float32),
        ],
    )(hidden_states, wt, b2, tri)
    return (sparse, logits)
